# baseline (device time: 884694 ns/iter reference)
import jax
import jax.numpy as jnp
from jax import lax
from jax.experimental import pallas as pl
from jax.experimental.pallas import tpu as pltpu

N_DEV = 16
N_EXPERTS = 64
CAP = 25
CAP_PAD = 32
E_LOCAL = N_EXPERTS // N_DEV
BLOCK = E_LOCAL * CAP_PAD


def _ring_allgather(compact_local):
    block, h = compact_local.shape

    def body(c_ref, out_ref, send_sems, recv_sems):
        my = lax.axis_index("i")
        right = (my + 1) % N_DEV

        out_ref[pl.ds(my, 1)] = c_ref[...][None, :, :]

        for k in range(N_DEV - 1):
            origin = (my + N_DEV - k) % N_DEV
            rdma = pltpu.make_async_remote_copy(
                src_ref=out_ref.at[origin],
                dst_ref=out_ref.at[origin],
                send_sem=send_sems.at[k],
                recv_sem=recv_sems.at[k],
                device_id=(right,),
                device_id_type=pl.DeviceIdType.MESH,
            )
            rdma.start()
            rdma.wait()

    return pl.pallas_call(
        body,
        out_shape=jax.ShapeDtypeStruct((N_DEV, block, h), compact_local.dtype),
        in_specs=[pl.BlockSpec(memory_space=pltpu.VMEM)],
        out_specs=pl.BlockSpec(memory_space=pltpu.VMEM),
        scratch_shapes=[
            pltpu.SemaphoreType.DMA((N_DEV - 1,)),
            pltpu.SemaphoreType.DMA((N_DEV - 1,)),
        ],
    )(compact_local)


def kernel(x, router_W, route_idx, expert_W):
    del router_W
    n_tok, d = x.shape
    h = expert_W.shape[-1]

    e = route_idx[:, 0]

    onehot = (e[:, None] == jnp.arange(N_EXPERTS, dtype=e.dtype)[None, :]).astype(
        jnp.int32
    )
    before = jnp.cumsum(onehot, axis=0) - onehot
    rank = jnp.take_along_axis(before, e[:, None].astype(jnp.int32), axis=1)[:, 0]
    accepted = rank < CAP

    slot = jnp.where(accepted, e.astype(jnp.int32) * CAP_PAD + rank, N_EXPERTS * CAP_PAD)
    token_of_slot = jnp.full((N_EXPERTS * CAP_PAD,), n_tok, dtype=jnp.int32)
    token_of_slot = token_of_slot.at[slot].set(
        jnp.arange(n_tok, dtype=jnp.int32), mode="drop"
    )

    my = lax.axis_index("i")
    my_slots = lax.dynamic_slice_in_dim(token_of_slot, my * BLOCK, BLOCK)

    x_pad = jnp.concatenate([x, jnp.zeros((1, d), x.dtype)], axis=0)
    xg = jnp.take(x_pad, my_slots, axis=0).reshape(E_LOCAL, CAP_PAD, d)
    compact_local = jnp.einsum(
        "ecd,edh->ech", xg, expert_W, preferred_element_type=jnp.float32
    ).reshape(BLOCK, h)

    compact_all = _ring_allgather(compact_local).reshape(N_DEV * BLOCK, h)

    compact_pad = jnp.concatenate([compact_all, jnp.zeros((1, h), jnp.float32)], axis=0)
    return jnp.take(compact_pad, slot, axis=0)


# device time: 151215 ns/iter; 5.8506x vs baseline; 5.8506x over previous
import jax
import jax.numpy as jnp
from jax import lax
from jax.experimental import pallas as pl
from jax.experimental.pallas import tpu as pltpu

N_DEV = 16
N_EXPERTS = 64
CAP = 25
CAP_PAD = 32
E_LOCAL = N_EXPERTS // N_DEV
BLOCK = E_LOCAL * CAP_PAD
SENTINEL = 4096


def _moe_body(slot_col_ref, slot_row_ref, x_ref, w_ref, out_ref, comm_ref,
              send_sems, recv_sems):
    n_tok = slot_col_ref.shape[0]
    my = lax.axis_index("i")
    right = lax.rem(my + 1, N_DEV)

    g_iota = lax.broadcasted_iota(jnp.int32, (BLOCK, n_tok), 0)
    G = (slot_row_ref[...] == my * BLOCK + g_iota).astype(jnp.float32)
    xg = jnp.dot(G, x_ref[...], preferred_element_type=jnp.float32)
    compact = jnp.concatenate(
        [
            jnp.dot(
                xg[e * CAP_PAD:(e + 1) * CAP_PAD, :],
                w_ref[e],
                preferred_element_type=jnp.float32,
            )
            for e in range(E_LOCAL)
        ],
        axis=0,
    )
    comm_ref[pl.ds(my, 1)] = compact[None, :, :]

    p_iota = lax.broadcasted_iota(jnp.int32, (n_tok, BLOCK), 1)

    def scatter_matmul(origin, block):
        P = (slot_col_ref[...] == origin * BLOCK + p_iota).astype(jnp.float32)
        return jnp.dot(P, block, preferred_element_type=jnp.float32)

    for k in range(N_DEV - 1):
        origin = lax.rem(my - k + N_DEV, N_DEV)
        rdma = pltpu.make_async_remote_copy(
            src_ref=comm_ref.at[origin],
            dst_ref=comm_ref.at[origin],
            send_sem=send_sems.at[k],
            recv_sem=recv_sems.at[k],
            device_id=(right,),
            device_id_type=pl.DeviceIdType.MESH,
        )
        rdma.start()
        if k == 0:
            out_ref[...] = scatter_matmul(my, compact)
        else:
            block = comm_ref[pl.ds(origin, 1)].reshape(BLOCK, -1)
            out_ref[...] += scatter_matmul(origin, block)
        rdma.wait()

    last = lax.rem(my + 1, N_DEV)
    block = comm_ref[pl.ds(last, 1)].reshape(BLOCK, -1)
    out_ref[...] += scatter_matmul(last, block)


def kernel(x, router_W, route_idx, expert_W):
    del router_W
    n_tok, d = x.shape
    h = expert_W.shape[-1]

    e = route_idx[:, 0].astype(jnp.int32)

    onehot = (e[:, None] == jnp.arange(N_EXPERTS, dtype=jnp.int32)[None, :]).astype(
        jnp.int32
    )
    before = jnp.cumsum(onehot, axis=0) - onehot
    rank = jnp.sum(before * onehot, axis=1)
    accepted = rank < CAP

    slot = jnp.where(accepted, e * CAP_PAD + rank, SENTINEL)

    return pl.pallas_call(
        _moe_body,
        out_shape=jax.ShapeDtypeStruct((n_tok, h), jnp.float32),
        in_specs=[pl.BlockSpec(memory_space=pltpu.VMEM)] * 4,
        out_specs=pl.BlockSpec(memory_space=pltpu.VMEM),
        scratch_shapes=[
            pltpu.VMEM((N_DEV, BLOCK, h), jnp.float32),
            pltpu.SemaphoreType.DMA((N_DEV - 1,)),
            pltpu.SemaphoreType.DMA((N_DEV - 1,)),
        ],
    )(slot[:, None], slot[None, :], x, expert_W)


# device time: 109434 ns/iter; 8.0843x vs baseline; 1.3818x over previous
import jax
import jax.numpy as jnp
from jax import lax
from jax.experimental import pallas as pl
from jax.experimental.pallas import tpu as pltpu

N_DEV = 16
N_EXPERTS = 64
CAP = 25
CAP_PAD = 32
E_LOCAL = N_EXPERTS // N_DEV
BLOCK = E_LOCAL * CAP_PAD
SENTINEL = 4096


def _moe_body(slot_col_ref, slot_row_ref, x_ref, w_ref, out_ref, comm_ref,
              send_sems, recv_sems):
    n_tok = slot_col_ref.shape[0]
    my = lax.axis_index("i")
    right = lax.rem(my + 1, N_DEV)

    g_iota = lax.broadcasted_iota(jnp.int32, (BLOCK, n_tok), 0)
    G = (slot_row_ref[...] == my * BLOCK + g_iota).astype(jnp.float32)
    xg = jnp.dot(G, x_ref[...], preferred_element_type=jnp.float32)
    compact = jnp.concatenate(
        [
            jnp.dot(
                xg[e * CAP_PAD:(e + 1) * CAP_PAD, :],
                w_ref[e],
                preferred_element_type=jnp.float32,
            )
            for e in range(E_LOCAL)
        ],
        axis=0,
    ).astype(jnp.bfloat16)
    comm_ref[pl.ds(my, 1)] = compact[None, :, :]

    p_iota = lax.broadcasted_iota(jnp.int32, (n_tok, BLOCK), 1)

    def scatter_matmul(origin, block):
        P = (slot_col_ref[...] == origin * BLOCK + p_iota).astype(jnp.bfloat16)
        return jnp.dot(P, block, preferred_element_type=jnp.float32)

    for k in range(N_DEV - 1):
        origin = lax.rem(my - k + N_DEV, N_DEV)
        rdma = pltpu.make_async_remote_copy(
            src_ref=comm_ref.at[origin],
            dst_ref=comm_ref.at[origin],
            send_sem=send_sems.at[k],
            recv_sem=recv_sems.at[k],
            device_id=(right,),
            device_id_type=pl.DeviceIdType.MESH,
        )
        rdma.start()
        if k == 0:
            out_ref[...] = scatter_matmul(my, compact)
        else:
            block = comm_ref[pl.ds(origin, 1)].reshape(BLOCK, -1)
            out_ref[...] += scatter_matmul(origin, block)
        rdma.wait()

    last = lax.rem(my + 1, N_DEV)
    block = comm_ref[pl.ds(last, 1)].reshape(BLOCK, -1)
    out_ref[...] += scatter_matmul(last, block)


def kernel(x, router_W, route_idx, expert_W):
    del router_W
    n_tok, d = x.shape
    h = expert_W.shape[-1]

    e = route_idx[:, 0].astype(jnp.int32)

    onehot = (e[:, None] == jnp.arange(N_EXPERTS, dtype=jnp.int32)[None, :]).astype(
        jnp.int32
    )
    before = jnp.cumsum(onehot, axis=0) - onehot
    rank = jnp.sum(before * onehot, axis=1)
    accepted = rank < CAP

    slot = jnp.where(accepted, e * CAP_PAD + rank, SENTINEL)

    return pl.pallas_call(
        _moe_body,
        out_shape=jax.ShapeDtypeStruct((n_tok, h), jnp.float32),
        in_specs=[pl.BlockSpec(memory_space=pltpu.VMEM)] * 4,
        out_specs=pl.BlockSpec(memory_space=pltpu.VMEM),
        scratch_shapes=[
            pltpu.VMEM((N_DEV, BLOCK, h), jnp.bfloat16),
            pltpu.SemaphoreType.DMA((N_DEV - 1,)),
            pltpu.SemaphoreType.DMA((N_DEV - 1,)),
        ],
    )(slot[:, None], slot[None, :], x, expert_W)


# device time: 85871 ns/iter; 10.3026x vs baseline; 1.2744x over previous
import jax
import jax.numpy as jnp
from jax import lax
from jax.experimental import pallas as pl
from jax.experimental.pallas import tpu as pltpu

N_DEV = 16
N_EXPERTS = 64
CAP = 25
CAP_PAD = 32
E_LOCAL = N_EXPERTS // N_DEV
BLOCK = E_LOCAL * CAP_PAD
SENTINEL = 4096
N_R = N_DEV // 2 - 1
N_L = N_DEV // 2


def _moe_body(slot_col_ref, slot_row_ref, x_ref, w_ref, out_ref, comm_ref,
              send_r, recv_r, send_l, recv_l):
    n_tok = slot_col_ref.shape[0]
    my = lax.axis_index("i")
    right = lax.rem(my + 1, N_DEV)
    left = lax.rem(my - 1 + N_DEV, N_DEV)

    g_iota = lax.broadcasted_iota(jnp.int32, (BLOCK, n_tok), 0)
    G = (slot_row_ref[...] == my * BLOCK + g_iota).astype(jnp.float32)
    xg = jnp.dot(G, x_ref[...], preferred_element_type=jnp.float32)
    compact = jnp.concatenate(
        [
            jnp.dot(
                xg[e * CAP_PAD:(e + 1) * CAP_PAD, :],
                w_ref[e],
                preferred_element_type=jnp.float32,
            )
            for e in range(E_LOCAL)
        ],
        axis=0,
    ).astype(jnp.bfloat16)
    comm_ref[pl.ds(my, 1)] = compact[None, :, :]

    p_iota = lax.broadcasted_iota(jnp.int32, (n_tok, BLOCK), 1)

    def scatter_matmul(origin, block):
        P = (slot_col_ref[...] == origin * BLOCK + p_iota).astype(jnp.bfloat16)
        return jnp.dot(P, block, preferred_element_type=jnp.float32)

    def load_block(origin):
        return comm_ref[pl.ds(origin, 1)].reshape(BLOCK, -1)

    for k in range(N_L):
        o_r = lax.rem(my - k + N_DEV, N_DEV)
        o_l = lax.rem(my + k, N_DEV)
        rdma_r = None
        if k < N_R:
            rdma_r = pltpu.make_async_remote_copy(
                src_ref=comm_ref.at[o_r],
                dst_ref=comm_ref.at[o_r],
                send_sem=send_r.at[k],
                recv_sem=recv_r.at[k],
                device_id=(right,),
                device_id_type=pl.DeviceIdType.MESH,
            )
            rdma_r.start()
        rdma_l = pltpu.make_async_remote_copy(
            src_ref=comm_ref.at[o_l],
            dst_ref=comm_ref.at[o_l],
            send_sem=send_l.at[k],
            recv_sem=recv_l.at[k],
            device_id=(left,),
            device_id_type=pl.DeviceIdType.MESH,
        )
        rdma_l.start()

        if k == 0:
            out_ref[...] = scatter_matmul(my, compact)
        else:
            out_ref[...] += scatter_matmul(o_r, load_block(o_r)) + scatter_matmul(
                o_l, load_block(o_l)
            )

        if rdma_r is not None:
            rdma_r.wait()
        rdma_l.wait()

    o_last = lax.rem(my + N_L, N_DEV)
    out_ref[...] += scatter_matmul(o_last, load_block(o_last))


def kernel(x, router_W, route_idx, expert_W):
    del router_W
    n_tok, d = x.shape
    h = expert_W.shape[-1]

    e = route_idx[:, 0].astype(jnp.int32)

    onehot = (e[:, None] == jnp.arange(N_EXPERTS, dtype=jnp.int32)[None, :]).astype(
        jnp.int32
    )
    before = jnp.cumsum(onehot, axis=0) - onehot
    rank = jnp.sum(before * onehot, axis=1)
    accepted = rank < CAP

    slot = jnp.where(accepted, e * CAP_PAD + rank, SENTINEL)

    return pl.pallas_call(
        _moe_body,
        out_shape=jax.ShapeDtypeStruct((n_tok, h), jnp.float32),
        in_specs=[pl.BlockSpec(memory_space=pltpu.VMEM)] * 4,
        out_specs=pl.BlockSpec(memory_space=pltpu.VMEM),
        scratch_shapes=[
            pltpu.VMEM((N_DEV, BLOCK, h), jnp.bfloat16),
            pltpu.SemaphoreType.DMA((N_R,)),
            pltpu.SemaphoreType.DMA((N_R,)),
            pltpu.SemaphoreType.DMA((N_L,)),
            pltpu.SemaphoreType.DMA((N_L,)),
        ],
    )(slot[:, None], slot[None, :], x, expert_W)
